# baseline (speedup 1.0000x reference)
import math

import jax
import jax.numpy as jnp
from jax import lax
from jax.experimental import pallas as pl
from jax.experimental.pallas import tpu as pltpu
from jax.experimental.pallas import tpu_sc as plsc

N = 10000
NP = 10240
E = 160000
EP = 163840
NC, NS, LN = 2, 16, 16
NT = NC * NS
CH = 128
DW = 128
F32 = jnp.float32
I32 = jnp.int32
NEG = -1e30

K1 = int(math.ceil(0.8 * N))
K2 = int(math.ceil(0.8 * K1))
K3 = int(math.ceil(0.8 * K2))


def _mesh():
    return plsc.VectorSubcoreMesh(core_axis_name="c", subcore_axis_name="s")


_SC_PARAMS = dict(compiler_params=pltpu.CompilerParams(needs_layout_passes=False))


def _sc_scalar_prop(scatter_by_row, has_w):
    EPT = EP // NT
    NCH = EPT // CH
    RP = NP // NS
    NB = 4

    def body(row_h, col_h, tab_h, *rest):
        if has_w:
            w_h, rest = rest[0], rest[1:]
        out_h = rest[0]
        tab_v, ri2_v, ci2_v, w2_v, val_v, z_v, acc_sh, ssem = rest[1:]
        c = lax.axis_index("c")
        s = lax.axis_index("s")
        wid = c * NS + s

        def zf(i, _):
            z_v[pl.ds(i * LN, LN)] = jnp.zeros((LN,), F32)
            return 0
        lax.fori_loop(0, RP // LN, zf, 0)
        pltpu.sync_copy(z_v, acc_sh.at[pl.ds(s * RP, RP)])
        plsc.subcore_barrier()

        pltpu.sync_copy(tab_h, tab_v)
        tb = wid * NCH
        pltpu.sync_copy(row_h.at[pl.ds(tb, NCH)], ri2_v)
        pltpu.sync_copy(col_h.at[pl.ds(tb, NCH)], ci2_v)
        if has_w:
            pltpu.sync_copy(w_h.at[pl.ds(tb, NCH)], w2_v)

        sd = [None] * NCH
        for k in range(NCH):
            b = k % NB
            if k >= NB:
                sd[k - NB].wait()
            for j in range(CH // LN):
                sl = pl.ds(j * LN, LN)
                if scatter_by_row:
                    v = (plsc.load_gather(tab_v, [ri2_v[k, sl]])
                         * plsc.load_gather(tab_v, [ci2_v[k, sl]]))
                else:
                    v = plsc.load_gather(tab_v, [ri2_v[k, sl]])
                    if has_w:
                        v = v * w2_v[k, sl]
                val_v[b, sl] = v
            idx_ref = ri2_v.at[k] if scatter_by_row else ci2_v.at[k]
            sd[k] = pltpu.async_copy(val_v.at[b], acc_sh.at[idx_ref],
                                     ssem.at[b], add=True)
        for k in range(NCH - NB, NCH):
            sd[k].wait()
        plsc.subcore_barrier()
        pltpu.sync_copy(acc_sh.at[pl.ds(s * RP, RP)],
                        out_h.at[pl.ds(c * NP + s * RP, RP)])

    return pl.kernel(
        body,
        out_type=jax.ShapeDtypeStruct((2 * NP,), F32),
        mesh=_mesh(),
        scratch_types=[
            pltpu.VMEM((NP,), F32),
            pltpu.VMEM((NCH, CH), I32),
            pltpu.VMEM((NCH, CH), I32),
            pltpu.VMEM((NCH, CH), F32),
            pltpu.VMEM((NB, CH), F32),
            pltpu.VMEM((RP,), F32),
            pltpu.VMEM_SHARED((NP,), F32),
            pltpu.SemaphoreType.DMA((NB,)),
        ],
        **_SC_PARAMS,
    )


def _sc_feat_prop(feat_split, has_w):
    EPT = EP // NS if feat_split else EP // NT
    NCH = EPT // CH
    RP = NP // NS

    CHF = 128
    EPTF = EP // NS if feat_split else EP // NT
    NCHF = EPTF // CHF
    NB = 2
    NG = NCHF // NB
    RPF = NP // NS

    def body(gs_h, row_h, col_h, *rest):
        if has_w:
            w_h, rest = rest[0], rest[1:]
        out_h = rest[0]
        ri_r, ci_r, w_r, rows_v, acc_sh, isem, gsem, ssem = rest[1:]
        c = lax.axis_index("c")
        s = lax.axis_index("s")
        coff = c * NP

        def zf(i, _):
            for t in range(DW // LN):
                rows_v[0, i, pl.ds(t * LN, LN)] = jnp.zeros((LN,), F32)
            return 0
        lax.fori_loop(0, CHF, zf, 0)
        zds = [pltpu.async_copy(rows_v.at[0],
                                acc_sh.at[pl.ds(s * RPF + m * CHF, CHF)],
                                gsem.at[m % NB])
               for m in range(RPF // CHF)]
        for d in zds:
            d.wait()
        plsc.subcore_barrier()

        tb = (s * NCHF) if feat_split else ((c * NS + s) * NCHF)

        def drain_scatters():
            for b in range(NB):
                pltpu.make_async_copy(gs_h.at[pl.ds(0, CHF)],
                                      rows_v.at[b], ssem.at[b]).wait()

        def rev(g, _):
            k0 = tb + g * NB
            @pl.when(g > 0)
            def _():
                drain_scatters()
            idescs = []
            for b in range(NB):
                idescs.append(pltpu.async_copy(row_h.at[k0 + b], ri_r.at[b],
                                               isem.at[b]))
                idescs.append(pltpu.async_copy(col_h.at[k0 + b], ci_r.at[b],
                                               isem.at[b]))
                if has_w:
                    idescs.append(pltpu.async_copy(w_h.at[k0 + b], w_r.at[b],
                                                   isem.at[b]))
            per = 3 if has_w else 2
            gdescs = []
            for b in range(NB):
                for d in idescs[b * per:(b + 1) * per]:
                    d.wait()
                if feat_split:
                    for j in range(CHF // LN):
                        sl = pl.ds(j * LN, LN)
                        ri_r[b, sl] = ri_r[b, sl] + coff
                gdescs.append(pltpu.async_copy(gs_h.at[ri_r.at[b]],
                                               rows_v.at[b], gsem.at[b]))
            for b in range(NB):
                gdescs[b].wait()
                if has_w:
                    def mulw(j, _, _b=b):
                        wj = plsc.load_gather(
                            w_r, [jnp.full((LN,), _b, I32),
                                  jnp.full((LN,), j, I32)])
                        for tt in range(DW // LN):
                            fs = pl.ds(tt * LN, LN)
                            rows_v[_b, j, fs] = rows_v[_b, j, fs] * wj
                        return 0
                    lax.fori_loop(0, CHF, mulw, 0)
                pltpu.async_copy(rows_v.at[b], acc_sh.at[ci_r.at[b]],
                                 ssem.at[b], add=True)
            return 0
        lax.fori_loop(0, NG, rev, 0)
        drain_scatters()
        plsc.subcore_barrier()
        pltpu.sync_copy(acc_sh.at[pl.ds(s * RPF, RPF)],
                        out_h.at[pl.ds(coff + s * RPF, RPF)])

    return pl.kernel(
        body,
        out_type=jax.ShapeDtypeStruct((2 * NP, DW), F32),
        mesh=_mesh(),
        scratch_types=[
            pltpu.VMEM((NB, CHF), I32),
            pltpu.VMEM((NB, CHF), I32),
            pltpu.VMEM((NB, CHF), F32),
            pltpu.VMEM((NB, CHF, DW), F32),
            pltpu.VMEM_SHARED((NP, DW), F32),
            pltpu.SemaphoreType.DMA((NB,)),
            pltpu.SemaphoreType.DMA((NB,)),
            pltpu.SemaphoreType.DMA((NB,)),
        ],
        **_SC_PARAMS,
    )


def _tc_mm(K, D, feat_split, BM=512):
    def body(x_ref, w_ref, d0, d1, ap, out_ref):
        degt = d0[...] + d1[...] + ap[...]
        dis = jnp.where(degt > 0, lax.rsqrt(degt), 0.0)
        g = jnp.dot(x_ref[...], w_ref[...], preferred_element_type=F32) * dis
        if feat_split:
            out_ref[0] = g[:, :DW]
            out_ref[1] = g[:, DW:]
        elif D == DW:
            out_ref[...] = g
        else:
            out_ref[...] = jnp.concatenate(
                [g, jnp.zeros((BM, DW - D), F32)], axis=1)

    if feat_split:
        out_spec = pl.BlockSpec((2, BM, DW), lambda i: (0, i, 0))
        out_shape = jax.ShapeDtypeStruct((2, NP, DW), F32)
    else:
        out_spec = pl.BlockSpec((BM, DW), lambda i: (i, 0))
        out_shape = jax.ShapeDtypeStruct((NP, DW), F32)
    return pl.pallas_call(
        body,
        grid=(NP // BM,),
        in_specs=[
            pl.BlockSpec((BM, K), lambda i: (i, 0)),
            pl.BlockSpec((K, D), lambda i: (0, 0)),
            pl.BlockSpec((BM, 1), lambda i: (i, 0)),
            pl.BlockSpec((BM, 1), lambda i: (i, 0)),
            pl.BlockSpec((BM, 1), lambda i: (i, 0)),
        ],
        out_specs=out_spec,
        out_shape=out_shape,
    )


def _tc_epi(D, feat_split, BM=512):
    def body(e_ref, gs_ref, d0, d1, ap, b_ref, th_ref, x_ref, ss_ref):
        degt = d0[...] + d1[...] + ap[...]
        dis = jnp.where(degt > 0, lax.rsqrt(degt), 0.0)
        if feat_split:
            esum = jnp.concatenate([e_ref[0], e_ref[1]], axis=1)
            gs = jnp.concatenate([gs_ref[0], gs_ref[1]], axis=1)
        else:
            esum = (e_ref[0] + e_ref[1])[:, :D]
            gs = gs_ref[...][:, :D]
        xv = jnp.maximum(dis * (esum + ap[...] * gs) + b_ref[...], 0.0) * ap[...]
        x_ref[...] = xv
        ss_ref[...] = jnp.dot(xv, th_ref[...], preferred_element_type=F32) * dis

    gs_spec = (pl.BlockSpec((2, BM, DW), lambda i: (0, i, 0)) if feat_split
               else pl.BlockSpec((BM, DW), lambda i: (i, 0)))
    return pl.pallas_call(
        body,
        grid=(NP // BM,),
        in_specs=[
            pl.BlockSpec((2, BM, DW), lambda i: (0, i, 0)),
            gs_spec,
            pl.BlockSpec((BM, 1), lambda i: (i, 0)),
            pl.BlockSpec((BM, 1), lambda i: (i, 0)),
            pl.BlockSpec((BM, 1), lambda i: (i, 0)),
            pl.BlockSpec((1, D), lambda i: (0, 0)),
            pl.BlockSpec((D, 1), lambda i: (0, 0)),
        ],
        out_specs=[
            pl.BlockSpec((BM, D), lambda i: (i, 0)),
            pl.BlockSpec((BM, 1), lambda i: (i, 0)),
        ],
        out_shape=[
            jax.ShapeDtypeStruct((NP, D), F32),
            jax.ShapeDtypeStruct((NP, 1), F32),
        ],
    )


def _tc_select(kk, D, last):
    def body(*refs):
        if last:
            (zp_ref, d0, d1, ap, ss, x_ref, ro1, ro2,
             wr1, br1, wr2, br2, wr3, br3, wo, bo,
             alive_ref, xp_ref, ro_ref, out_ref) = refs
        else:
            (zp_ref, d0, d1, ap, ss, x_ref,
             alive_ref, xp_ref, ro_ref) = refs
        degt = d0[...] + d1[...] + ap[...]
        dis = jnp.where(degt > 0, lax.rsqrt(degt), 0.0)
        z = dis * (zp_ref[0] + zp_ref[1] + ap[...] * ss[...])
        zkey = jnp.where(ap[...] > 0, z, NEG)
        ki = lax.bitcast_convert_type(zkey, I32)
        ks = jnp.where(ki < 0, ki ^ jnp.int32(0x7FFFFFFF), ki)

        def bstep(t, pre):
            cand = pre + lax.shift_left(jnp.int32(1), jnp.int32(31) - t)
            cnt = jnp.sum((ks >= cand).astype(I32))
            return jnp.where(cnt >= kk, cand, pre)
        T = lax.fori_loop(0, 32, bstep, jnp.int32(-2147483648))

        gt = ks > T
        need = jnp.int32(kk) - jnp.sum(gt.astype(I32))
        tied = ks == T
        idx = (lax.broadcasted_iota(I32, (80, 128), 0) * 128
               + lax.broadcasted_iota(I32, (80, 128), 1))

        def tstep(t, pre):
            cand = pre + lax.shift_left(jnp.int32(1), jnp.int32(13) - t)
            cnt = jnp.sum((tied & (idx < cand)).astype(I32))
            return jnp.where(cnt < need, cand, pre)
        C0 = lax.fori_loop(0, 14, tstep, jnp.int32(0))

        alive = gt | (tied & (idx < (C0 + 1)) & (need > 0))
        a = alive.astype(F32)
        alive_ref[...] = a
        gate = a * jnp.tanh(z)
        xp = x_ref[...] * gate[..., None]
        xp_ref[...] = xp
        ro = jnp.sum(xp, axis=(0, 1)).reshape(1, D)
        ro_ref[...] = ro
        if last:
            r = (jnp.dot(ro1[...], wr1[...], preferred_element_type=F32) + br1[...]
                 + jnp.dot(ro2[...], wr2[...], preferred_element_type=F32) + br2[...]
                 + jnp.dot(ro, wr3[...], preferred_element_type=F32) + br3[...])
            out_ref[...] = (jnp.dot(jnp.maximum(r, 0.0), wo[...],
                                    preferred_element_type=F32) + bo[...])

    outs = [
        jax.ShapeDtypeStruct((80, 128), F32),
        jax.ShapeDtypeStruct((80, 128, D), F32),
        jax.ShapeDtypeStruct((1, D), F32),
    ]
    if last:
        outs.append(jax.ShapeDtypeStruct((1, 128), F32))
    return pl.pallas_call(body, out_shape=outs)


def kernel(x, edge_index, edge_attr, batch, W1, b1, theta1, W2, b2, theta2,
           W3, b3, theta3, Wr1, br1, Wr2, br2, Wr3, br3, Wout, bout):
    del batch
    row = edge_index[0]
    col = edge_index[1]
    pe = EP - E
    rowf = jnp.concatenate([row, jnp.full((pe,), NP - 1, I32)])
    colf = jnp.concatenate([col, jnp.full((pe,), NP - 1, I32)])
    wf = jnp.concatenate([edge_attr.astype(F32), jnp.zeros((pe,), F32)])
    rowp = rowf.reshape(EP // CH, CH)
    colp = colf.reshape(EP // CH, CH)
    wp = wf.reshape(EP // CH, CH)
    rowp64 = rowf.reshape(EP // 64, 64)
    colp64 = colf.reshape(EP // 64, 64)
    wp64 = wf.reshape(EP // 64, 64)
    xpad = jnp.pad(x, ((0, NP - N), (0, 0)))
    alive0 = jnp.concatenate([jnp.ones((N,), F32), jnp.zeros((NP - N,), F32)])

    deg_k = _sc_scalar_prop(True, False)
    zprop_w = _sc_scalar_prop(False, True)
    zprop = _sc_scalar_prop(False, False)

    def layer(inp, alive, K, D, W, b, theta, kk, first, last, head=None):
        feat_split = first
        acol = alive.reshape(NP, 1)
        a2d = alive.reshape(80, 128)
        degp = deg_k(rowp, colp, alive).reshape(2, NP)
        d0 = degp[0].reshape(NP, 1)
        d1 = degp[1].reshape(NP, 1)
        d02 = degp[0].reshape(80, 128)
        d12 = degp[1].reshape(80, 128)
        gs = _tc_mm(K, D, feat_split)(inp, W, d0, d1, acol)
        feat = _sc_feat_prop(feat_split, first)
        gs_sc = gs.reshape(2 * NP, DW) if feat_split else gs
        if first:
            es = feat(gs_sc, rowp, colp, wp)
        else:
            es = feat(gs_sc, rowp, colp)
        xv, ss = _tc_epi(D, feat_split)(es.reshape(2, NP, DW), gs, d0, d1,
                                        acol, b.reshape(1, D), theta)
        if first:
            zp = zprop_w(rowp, colp, ss.reshape(NP), wp)
        else:
            zp = zprop(rowp, colp, ss.reshape(NP))
        sel_in = [zp.reshape(2, 80, 128), d02, d12, a2d,
                  ss.reshape(80, 128), xv.reshape(80, 128, D)]
        if last:
            sel_in += head
            return _tc_select(kk, D, True)(*sel_in)
        return _tc_select(kk, D, False)(*sel_in)

    a1, x1p, ro1 = layer(xpad, alive0, 256, 256, W1, b1, theta1,
                         K1, True, False)
    a2, x2p, ro2 = layer(x1p.reshape(NP, 256), a1.reshape(NP), 256, 128,
                         W2, b2, theta2, K2, False, False)
    head = [ro1, ro2, Wr1, br1.reshape(1, 64), Wr2, br2.reshape(1, 64),
            Wr3, br3.reshape(1, 64), Wout, bout.reshape(1, 128)]
    _, _, _, out = layer(x2p.reshape(NP, 128), a2.reshape(NP), 128, 64,
                         W3, b3, theta3, K3, False, True, head)
    return out


# speedup vs baseline: 1.1085x; 1.1085x over previous
import math

import jax
import jax.numpy as jnp
from jax import lax
from jax.experimental import pallas as pl
from jax.experimental.pallas import tpu as pltpu
from jax.experimental.pallas import tpu_sc as plsc

N = 10000
NP = 10240
E = 160000
EP = 163840
NC, NS, LN = 2, 16, 16
NT = NC * NS
CH = 128
DW = 128
F32 = jnp.float32
I32 = jnp.int32
NEG = -1e30

K1 = int(math.ceil(0.8 * N))
K2 = int(math.ceil(0.8 * K1))
K3 = int(math.ceil(0.8 * K2))


def _mesh():
    return plsc.VectorSubcoreMesh(core_axis_name="c", subcore_axis_name="s")


_SC_PARAMS = dict(compiler_params=pltpu.CompilerParams(needs_layout_passes=False))


def _sc_scalar_prop(scatter_by_row, has_w):
    EPT = EP // NT
    NCH = EPT // CH
    RP = NP // NS
    NB = 4

    def body(row_h, col_h, tab_h, *rest):
        if has_w:
            w_h, rest = rest[0], rest[1:]
        out_h = rest[0]
        tab_v, ri2_v, ci2_v, w2_v, val_v, z_v, acc_sh, ssem = rest[1:]
        c = lax.axis_index("c")
        s = lax.axis_index("s")
        wid = c * NS + s

        def zf(i, _):
            z_v[pl.ds(i * LN, LN)] = jnp.zeros((LN,), F32)
            return 0
        lax.fori_loop(0, RP // LN, zf, 0)
        pltpu.sync_copy(z_v, acc_sh.at[pl.ds(s * RP, RP)])
        plsc.subcore_barrier()

        pltpu.sync_copy(tab_h, tab_v)
        tb = wid * NCH
        pltpu.sync_copy(row_h.at[pl.ds(tb, NCH)], ri2_v)
        pltpu.sync_copy(col_h.at[pl.ds(tb, NCH)], ci2_v)
        if has_w:
            pltpu.sync_copy(w_h.at[pl.ds(tb, NCH)], w2_v)

        sd = [None] * NCH
        for k in range(NCH):
            b = k % NB
            if k >= NB:
                sd[k - NB].wait()
            for j in range(CH // LN):
                sl = pl.ds(j * LN, LN)
                if scatter_by_row:
                    v = (plsc.load_gather(tab_v, [ri2_v[k, sl]])
                         * plsc.load_gather(tab_v, [ci2_v[k, sl]]))
                else:
                    v = plsc.load_gather(tab_v, [ri2_v[k, sl]])
                    if has_w:
                        v = v * w2_v[k, sl]
                val_v[b, sl] = v
            idx_ref = ri2_v.at[k] if scatter_by_row else ci2_v.at[k]
            sd[k] = pltpu.async_copy(val_v.at[b], acc_sh.at[idx_ref],
                                     ssem.at[b], add=True)
        for k in range(NCH - NB, NCH):
            sd[k].wait()
        plsc.subcore_barrier()
        pltpu.sync_copy(acc_sh.at[pl.ds(s * RP, RP)],
                        out_h.at[pl.ds(c * NP + s * RP, RP)])

    return pl.kernel(
        body,
        out_type=jax.ShapeDtypeStruct((2 * NP,), F32),
        mesh=_mesh(),
        scratch_types=[
            pltpu.VMEM((NP,), F32),
            pltpu.VMEM((NCH, CH), I32),
            pltpu.VMEM((NCH, CH), I32),
            pltpu.VMEM((NCH, CH), F32),
            pltpu.VMEM((NB, CH), F32),
            pltpu.VMEM((RP,), F32),
            pltpu.VMEM_SHARED((NP,), F32),
            pltpu.SemaphoreType.DMA((NB,)),
        ],
        **_SC_PARAMS,
    )


def _sc_feat_prop(feat_split, has_w):
    EPT = EP // NS if feat_split else EP // NT
    NCH = EPT // CH
    RP = NP // NS

    CHF = 32
    EPTF = EP // NS if feat_split else EP // NT
    NCHF = EPTF // CHF
    NB = 8
    NG = NCHF // NB
    RPF = NP // NS

    def body(gs_h, row_h, col_h, *rest):
        if has_w:
            w_h, rest = rest[0], rest[1:]
        out_h = rest[0]
        ri_r, ci_r, w_r, rows_v, acc_sh, isem, gsem, ssem = rest[1:]
        c = lax.axis_index("c")
        s = lax.axis_index("s")
        coff = c * NP

        def zf(i, _):
            for t in range(DW // LN):
                rows_v[0, i, pl.ds(t * LN, LN)] = jnp.zeros((LN,), F32)
            return 0
        lax.fori_loop(0, CHF, zf, 0)
        zds = [pltpu.async_copy(rows_v.at[0],
                                acc_sh.at[pl.ds(s * RPF + m * CHF, CHF)],
                                gsem.at[m % NB])
               for m in range(RPF // CHF)]
        for d in zds:
            d.wait()
        plsc.subcore_barrier()

        tb = (s * NCHF) if feat_split else ((c * NS + s) * NCHF)

        def drain_scatters():
            for b in range(NB):
                pltpu.make_async_copy(gs_h.at[pl.ds(0, CHF)],
                                      rows_v.at[b], ssem.at[b]).wait()

        def rev(g, _):
            k0 = tb + g * NB
            @pl.when(g > 0)
            def _():
                drain_scatters()
            idescs = []
            for b in range(NB):
                idescs.append(pltpu.async_copy(row_h.at[k0 + b], ri_r.at[b],
                                               isem.at[b]))
                idescs.append(pltpu.async_copy(col_h.at[k0 + b], ci_r.at[b],
                                               isem.at[b]))
                if has_w:
                    idescs.append(pltpu.async_copy(w_h.at[k0 + b], w_r.at[b],
                                                   isem.at[b]))
            per = 3 if has_w else 2
            gdescs = []
            for b in range(NB):
                for d in idescs[b * per:(b + 1) * per]:
                    d.wait()
                if feat_split:
                    for j in range(CHF // LN):
                        sl = pl.ds(j * LN, LN)
                        ri_r[b, sl] = ri_r[b, sl] + coff
                gdescs.append(pltpu.async_copy(gs_h.at[ri_r.at[b]],
                                               rows_v.at[b], gsem.at[b]))
            for b in range(NB):
                gdescs[b].wait()
                if has_w:
                    def mulw(j, _, _b=b):
                        wj = plsc.load_gather(
                            w_r, [jnp.full((LN,), _b, I32),
                                  jnp.full((LN,), j, I32)])
                        for tt in range(DW // LN):
                            fs = pl.ds(tt * LN, LN)
                            rows_v[_b, j, fs] = rows_v[_b, j, fs] * wj
                        return 0
                    lax.fori_loop(0, CHF, mulw, 0)
                pltpu.async_copy(rows_v.at[b], acc_sh.at[ci_r.at[b]],
                                 ssem.at[b], add=True)
            return 0
        lax.fori_loop(0, NG, rev, 0)
        drain_scatters()
        plsc.subcore_barrier()
        pltpu.sync_copy(acc_sh.at[pl.ds(s * RPF, RPF)],
                        out_h.at[pl.ds(coff + s * RPF, RPF)])

    return pl.kernel(
        body,
        out_type=jax.ShapeDtypeStruct((2 * NP, DW), F32),
        mesh=_mesh(),
        scratch_types=[
            pltpu.VMEM((NB, CHF), I32),
            pltpu.VMEM((NB, CHF), I32),
            pltpu.VMEM((NB, CHF), F32),
            pltpu.VMEM((NB, CHF, DW), F32),
            pltpu.VMEM_SHARED((NP, DW), F32),
            pltpu.SemaphoreType.DMA((NB,)),
            pltpu.SemaphoreType.DMA((NB,)),
            pltpu.SemaphoreType.DMA((NB,)),
        ],
        **_SC_PARAMS,
    )


def _tc_mm(K, D, feat_split, BM=512):
    def body(x_ref, w_ref, d0, d1, ap, out_ref):
        degt = d0[...] + d1[...] + ap[...]
        dis = jnp.where(degt > 0, lax.rsqrt(degt), 0.0)
        g = jnp.dot(x_ref[...], w_ref[...], preferred_element_type=F32) * dis
        if feat_split:
            out_ref[0] = g[:, :DW]
            out_ref[1] = g[:, DW:]
        elif D == DW:
            out_ref[...] = g
        else:
            out_ref[...] = jnp.concatenate(
                [g, jnp.zeros((BM, DW - D), F32)], axis=1)

    if feat_split:
        out_spec = pl.BlockSpec((2, BM, DW), lambda i: (0, i, 0))
        out_shape = jax.ShapeDtypeStruct((2, NP, DW), F32)
    else:
        out_spec = pl.BlockSpec((BM, DW), lambda i: (i, 0))
        out_shape = jax.ShapeDtypeStruct((NP, DW), F32)
    return pl.pallas_call(
        body,
        grid=(NP // BM,),
        in_specs=[
            pl.BlockSpec((BM, K), lambda i: (i, 0)),
            pl.BlockSpec((K, D), lambda i: (0, 0)),
            pl.BlockSpec((BM, 1), lambda i: (i, 0)),
            pl.BlockSpec((BM, 1), lambda i: (i, 0)),
            pl.BlockSpec((BM, 1), lambda i: (i, 0)),
        ],
        out_specs=out_spec,
        out_shape=out_shape,
    )


def _tc_epi(D, feat_split, BM=512):
    def body(e_ref, gs_ref, d0, d1, ap, b_ref, th_ref, x_ref, ss_ref):
        degt = d0[...] + d1[...] + ap[...]
        dis = jnp.where(degt > 0, lax.rsqrt(degt), 0.0)
        if feat_split:
            esum = jnp.concatenate([e_ref[0], e_ref[1]], axis=1)
            gs = jnp.concatenate([gs_ref[0], gs_ref[1]], axis=1)
        else:
            esum = (e_ref[0] + e_ref[1])[:, :D]
            gs = gs_ref[...][:, :D]
        xv = jnp.maximum(dis * (esum + ap[...] * gs) + b_ref[...], 0.0) * ap[...]
        x_ref[...] = xv
        ss_ref[...] = jnp.dot(xv, th_ref[...], preferred_element_type=F32) * dis

    gs_spec = (pl.BlockSpec((2, BM, DW), lambda i: (0, i, 0)) if feat_split
               else pl.BlockSpec((BM, DW), lambda i: (i, 0)))
    return pl.pallas_call(
        body,
        grid=(NP // BM,),
        in_specs=[
            pl.BlockSpec((2, BM, DW), lambda i: (0, i, 0)),
            gs_spec,
            pl.BlockSpec((BM, 1), lambda i: (i, 0)),
            pl.BlockSpec((BM, 1), lambda i: (i, 0)),
            pl.BlockSpec((BM, 1), lambda i: (i, 0)),
            pl.BlockSpec((1, D), lambda i: (0, 0)),
            pl.BlockSpec((D, 1), lambda i: (0, 0)),
        ],
        out_specs=[
            pl.BlockSpec((BM, D), lambda i: (i, 0)),
            pl.BlockSpec((BM, 1), lambda i: (i, 0)),
        ],
        out_shape=[
            jax.ShapeDtypeStruct((NP, D), F32),
            jax.ShapeDtypeStruct((NP, 1), F32),
        ],
    )


def _tc_select(kk, D, last):
    def body(*refs):
        if last:
            (zp_ref, d0, d1, ap, ss, x_ref, ro1, ro2,
             wr1, br1, wr2, br2, wr3, br3, wo, bo,
             alive_ref, xp_ref, ro_ref, out_ref) = refs
        else:
            (zp_ref, d0, d1, ap, ss, x_ref,
             alive_ref, xp_ref, ro_ref) = refs
        degt = d0[...] + d1[...] + ap[...]
        dis = jnp.where(degt > 0, lax.rsqrt(degt), 0.0)
        z = dis * (zp_ref[0] + zp_ref[1] + ap[...] * ss[...])
        zkey = jnp.where(ap[...] > 0, z, NEG)
        ki = lax.bitcast_convert_type(zkey, I32)
        ks = jnp.where(ki < 0, ki ^ jnp.int32(0x7FFFFFFF), ki)

        def bstep(t, pre):
            cand = pre + lax.shift_left(jnp.int32(1), jnp.int32(31) - t)
            cnt = jnp.sum((ks >= cand).astype(I32))
            return jnp.where(cnt >= kk, cand, pre)
        T = lax.fori_loop(0, 32, bstep, jnp.int32(-2147483648))

        gt = ks > T
        need = jnp.int32(kk) - jnp.sum(gt.astype(I32))
        tied = ks == T
        idx = (lax.broadcasted_iota(I32, (80, 128), 0) * 128
               + lax.broadcasted_iota(I32, (80, 128), 1))

        def tstep(t, pre):
            cand = pre + lax.shift_left(jnp.int32(1), jnp.int32(13) - t)
            cnt = jnp.sum((tied & (idx < cand)).astype(I32))
            return jnp.where(cnt < need, cand, pre)
        C0 = lax.fori_loop(0, 14, tstep, jnp.int32(0))

        alive = gt | (tied & (idx < (C0 + 1)) & (need > 0))
        a = alive.astype(F32)
        alive_ref[...] = a
        gate = a * jnp.tanh(z)
        xp = x_ref[...] * gate[..., None]
        xp_ref[...] = xp
        ro = jnp.sum(xp, axis=(0, 1)).reshape(1, D)
        ro_ref[...] = ro
        if last:
            r = (jnp.dot(ro1[...], wr1[...], preferred_element_type=F32) + br1[...]
                 + jnp.dot(ro2[...], wr2[...], preferred_element_type=F32) + br2[...]
                 + jnp.dot(ro, wr3[...], preferred_element_type=F32) + br3[...])
            out_ref[...] = (jnp.dot(jnp.maximum(r, 0.0), wo[...],
                                    preferred_element_type=F32) + bo[...])

    outs = [
        jax.ShapeDtypeStruct((80, 128), F32),
        jax.ShapeDtypeStruct((80, 128, D), F32),
        jax.ShapeDtypeStruct((1, D), F32),
    ]
    if last:
        outs.append(jax.ShapeDtypeStruct((1, 128), F32))
    return pl.pallas_call(body, out_shape=outs)


def kernel(x, edge_index, edge_attr, batch, W1, b1, theta1, W2, b2, theta2,
           W3, b3, theta3, Wr1, br1, Wr2, br2, Wr3, br3, Wout, bout):
    del batch
    row = edge_index[0]
    col = edge_index[1]
    pe = EP - E
    rowf = jnp.concatenate([row, jnp.full((pe,), NP - 1, I32)])
    colf = jnp.concatenate([col, jnp.full((pe,), NP - 1, I32)])
    wf = jnp.concatenate([edge_attr.astype(F32), jnp.zeros((pe,), F32)])
    rowp = rowf.reshape(EP // CH, CH)
    colp = colf.reshape(EP // CH, CH)
    wp = wf.reshape(EP // CH, CH)
    rowp64 = rowf.reshape(EP // 32, 32)
    colp64 = colf.reshape(EP // 32, 32)
    wp64 = wf.reshape(EP // 32, 32)
    xpad = jnp.pad(x, ((0, NP - N), (0, 0)))
    alive0 = jnp.concatenate([jnp.ones((N,), F32), jnp.zeros((NP - N,), F32)])

    deg_k = _sc_scalar_prop(True, False)
    zprop_w = _sc_scalar_prop(False, True)
    zprop = _sc_scalar_prop(False, False)

    def layer(inp, alive, K, D, W, b, theta, kk, first, last, head=None):
        feat_split = first
        acol = alive.reshape(NP, 1)
        a2d = alive.reshape(80, 128)
        degp = deg_k(rowp, colp, alive).reshape(2, NP)
        d0 = degp[0].reshape(NP, 1)
        d1 = degp[1].reshape(NP, 1)
        d02 = degp[0].reshape(80, 128)
        d12 = degp[1].reshape(80, 128)
        gs = _tc_mm(K, D, feat_split)(inp, W, d0, d1, acol)
        feat = _sc_feat_prop(feat_split, first)
        gs_sc = gs.reshape(2 * NP, DW) if feat_split else gs
        if first:
            es = feat(gs_sc, rowp64, colp64, wp64)
        else:
            es = feat(gs_sc, rowp64, colp64)
        xv, ss = _tc_epi(D, feat_split)(es.reshape(2, NP, DW), gs, d0, d1,
                                        acol, b.reshape(1, D), theta)
        if first:
            zp = zprop_w(rowp, colp, ss.reshape(NP), wp)
        else:
            zp = zprop(rowp, colp, ss.reshape(NP))
        sel_in = [zp.reshape(2, 80, 128), d02, d12, a2d,
                  ss.reshape(80, 128), xv.reshape(80, 128, D)]
        if last:
            sel_in += head
            return _tc_select(kk, D, True)(*sel_in)
        return _tc_select(kk, D, False)(*sel_in)

    a1, x1p, ro1 = layer(xpad, alive0, 256, 256, W1, b1, theta1,
                         K1, True, False)
    a2, x2p, ro2 = layer(x1p.reshape(NP, 256), a1.reshape(NP), 256, 128,
                         W2, b2, theta2, K2, False, False)
    head = [ro1, ro2, Wr1, br1.reshape(1, 64), Wr2, br2.reshape(1, 64),
            Wr3, br3.reshape(1, 64), Wout, bout.reshape(1, 128)]
    _, _, _, out = layer(x2p.reshape(NP, 128), a2.reshape(NP), 128, 64,
                         W3, b3, theta3, K3, False, True, head)
    return out


# speedup vs baseline: 1.1613x; 1.0477x over previous
import math

import jax
import jax.numpy as jnp
from jax import lax
from jax.experimental import pallas as pl
from jax.experimental.pallas import tpu as pltpu
from jax.experimental.pallas import tpu_sc as plsc

N = 10000
NP = 10240
E = 160000
EP = 163840
NC, NS, LN = 2, 16, 16
NT = NC * NS
CH = 128
DW = 128
F32 = jnp.float32
I32 = jnp.int32
NEG = -1e30

K1 = int(math.ceil(0.8 * N))
K2 = int(math.ceil(0.8 * K1))
K3 = int(math.ceil(0.8 * K2))


def _mesh():
    return plsc.VectorSubcoreMesh(core_axis_name="c", subcore_axis_name="s")


_SC_PARAMS = dict(compiler_params=pltpu.CompilerParams(needs_layout_passes=False))


def _sc_scalar_prop(scatter_by_row, has_w):
    EPT = EP // NT
    NCH = EPT // CH
    RP = NP // NS
    NB = 4

    def body(row_h, col_h, tab_h, *rest):
        if has_w:
            w_h, rest = rest[0], rest[1:]
        out_h = rest[0]
        tab_v, ri2_v, ci2_v, w2_v, val_v, z_v, acc_sh, ssem = rest[1:]
        c = lax.axis_index("c")
        s = lax.axis_index("s")
        wid = c * NS + s

        def zf(i, _):
            z_v[pl.ds(i * LN, LN)] = jnp.zeros((LN,), F32)
            return 0
        lax.fori_loop(0, RP // LN, zf, 0)
        pltpu.sync_copy(z_v, acc_sh.at[pl.ds(s * RP, RP)])
        plsc.subcore_barrier()

        pltpu.sync_copy(tab_h, tab_v)
        tb = wid * NCH
        pltpu.sync_copy(row_h.at[pl.ds(tb, NCH)], ri2_v)
        pltpu.sync_copy(col_h.at[pl.ds(tb, NCH)], ci2_v)
        if has_w:
            pltpu.sync_copy(w_h.at[pl.ds(tb, NCH)], w2_v)

        sd = [None] * NCH
        for k in range(NCH):
            b = k % NB
            if k >= NB:
                sd[k - NB].wait()
            for j in range(CH // LN):
                sl = pl.ds(j * LN, LN)
                if scatter_by_row:
                    v = (plsc.load_gather(tab_v, [ri2_v[k, sl]])
                         * plsc.load_gather(tab_v, [ci2_v[k, sl]]))
                else:
                    v = plsc.load_gather(tab_v, [ri2_v[k, sl]])
                    if has_w:
                        v = v * w2_v[k, sl]
                val_v[b, sl] = v
            idx_ref = ri2_v.at[k] if scatter_by_row else ci2_v.at[k]
            sd[k] = pltpu.async_copy(val_v.at[b], acc_sh.at[idx_ref],
                                     ssem.at[b], add=True)
        for k in range(NCH - NB, NCH):
            sd[k].wait()
        plsc.subcore_barrier()
        pltpu.sync_copy(acc_sh.at[pl.ds(s * RP, RP)],
                        out_h.at[pl.ds(c * NP + s * RP, RP)])

    return pl.kernel(
        body,
        out_type=jax.ShapeDtypeStruct((2 * NP,), F32),
        mesh=_mesh(),
        scratch_types=[
            pltpu.VMEM((NP,), F32),
            pltpu.VMEM((NCH, CH), I32),
            pltpu.VMEM((NCH, CH), I32),
            pltpu.VMEM((NCH, CH), F32),
            pltpu.VMEM((NB, CH), F32),
            pltpu.VMEM((RP,), F32),
            pltpu.VMEM_SHARED((NP,), F32),
            pltpu.SemaphoreType.DMA((NB,)),
        ],
        **_SC_PARAMS,
    )


def _sc_feat_prop(feat_split, has_w):
    EPT = EP // NS if feat_split else EP // NT
    NCH = EPT // CH
    RP = NP // NS

    CHF = 64
    EPTF = EP // NS if feat_split else EP // NT
    NCHF = EPTF // CHF
    NB = 5
    NG = NCHF // NB
    RPF = NP // NS

    def body(gs_h, row_h, col_h, *rest):
        if has_w:
            w_h, rest = rest[0], rest[1:]
        out_h = rest[0]
        ri_r, ci_r, w_r, rows_v, acc_sh, isem, gsem, ssem = rest[1:]
        c = lax.axis_index("c")
        s = lax.axis_index("s")
        coff = c * NP

        def zf(i, _):
            for t in range(DW // LN):
                rows_v[0, i, pl.ds(t * LN, LN)] = jnp.zeros((LN,), F32)
            return 0
        lax.fori_loop(0, CHF, zf, 0)
        zds = [pltpu.async_copy(rows_v.at[0],
                                acc_sh.at[pl.ds(s * RPF + m * CHF, CHF)],
                                gsem.at[m % NB])
               for m in range(RPF // CHF)]
        for d in zds:
            d.wait()
        plsc.subcore_barrier()

        tb = (s * NCHF) if feat_split else ((c * NS + s) * NCHF)

        def drain_scatters():
            for b in range(NB):
                pltpu.make_async_copy(gs_h.at[pl.ds(0, CHF)],
                                      rows_v.at[b], ssem.at[b]).wait()

        def rev(g, _):
            k0 = tb + g * NB
            @pl.when(g > 0)
            def _():
                drain_scatters()
            idescs = []
            for b in range(NB):
                idescs.append(pltpu.async_copy(row_h.at[k0 + b], ri_r.at[b],
                                               isem.at[b]))
                idescs.append(pltpu.async_copy(col_h.at[k0 + b], ci_r.at[b],
                                               isem.at[b]))
                if has_w:
                    idescs.append(pltpu.async_copy(w_h.at[k0 + b], w_r.at[b],
                                                   isem.at[b]))
            per = 3 if has_w else 2
            gdescs = []
            for b in range(NB):
                for d in idescs[b * per:(b + 1) * per]:
                    d.wait()
                if feat_split:
                    for j in range(CHF // LN):
                        sl = pl.ds(j * LN, LN)
                        ri_r[b, sl] = ri_r[b, sl] + coff
                gdescs.append(pltpu.async_copy(gs_h.at[ri_r.at[b]],
                                               rows_v.at[b], gsem.at[b]))
            for b in range(NB):
                gdescs[b].wait()
                if has_w:
                    def mulw(j, _, _b=b):
                        wj = plsc.load_gather(
                            w_r, [jnp.full((LN,), _b, I32),
                                  jnp.full((LN,), j, I32)])
                        for tt in range(DW // LN):
                            fs = pl.ds(tt * LN, LN)
                            rows_v[_b, j, fs] = rows_v[_b, j, fs] * wj
                        return 0
                    lax.fori_loop(0, CHF, mulw, 0)
                pltpu.async_copy(rows_v.at[b], acc_sh.at[ci_r.at[b]],
                                 ssem.at[b], add=True)
            return 0
        lax.fori_loop(0, NG, rev, 0)
        drain_scatters()
        plsc.subcore_barrier()
        pltpu.sync_copy(acc_sh.at[pl.ds(s * RPF, RPF)],
                        out_h.at[pl.ds(coff + s * RPF, RPF)])

    return pl.kernel(
        body,
        out_type=jax.ShapeDtypeStruct((2 * NP, DW), F32),
        mesh=_mesh(),
        scratch_types=[
            pltpu.VMEM((NB, CHF), I32),
            pltpu.VMEM((NB, CHF), I32),
            pltpu.VMEM((NB, CHF), F32),
            pltpu.VMEM((NB, CHF, DW), F32),
            pltpu.VMEM_SHARED((NP, DW), F32),
            pltpu.SemaphoreType.DMA((NB,)),
            pltpu.SemaphoreType.DMA((NB,)),
            pltpu.SemaphoreType.DMA((NB,)),
        ],
        **_SC_PARAMS,
    )


def _tc_mm(K, D, feat_split, BM=512):
    def body(x_ref, w_ref, d0, d1, ap, out_ref):
        degt = d0[...] + d1[...] + ap[...]
        dis = jnp.where(degt > 0, lax.rsqrt(degt), 0.0)
        g = jnp.dot(x_ref[...], w_ref[...], preferred_element_type=F32) * dis
        if feat_split:
            out_ref[0] = g[:, :DW]
            out_ref[1] = g[:, DW:]
        elif D == DW:
            out_ref[...] = g
        else:
            out_ref[...] = jnp.concatenate(
                [g, jnp.zeros((BM, DW - D), F32)], axis=1)

    if feat_split:
        out_spec = pl.BlockSpec((2, BM, DW), lambda i: (0, i, 0))
        out_shape = jax.ShapeDtypeStruct((2, NP, DW), F32)
    else:
        out_spec = pl.BlockSpec((BM, DW), lambda i: (i, 0))
        out_shape = jax.ShapeDtypeStruct((NP, DW), F32)
    return pl.pallas_call(
        body,
        grid=(NP // BM,),
        in_specs=[
            pl.BlockSpec((BM, K), lambda i: (i, 0)),
            pl.BlockSpec((K, D), lambda i: (0, 0)),
            pl.BlockSpec((BM, 1), lambda i: (i, 0)),
            pl.BlockSpec((BM, 1), lambda i: (i, 0)),
            pl.BlockSpec((BM, 1), lambda i: (i, 0)),
        ],
        out_specs=out_spec,
        out_shape=out_shape,
    )


def _tc_epi(D, feat_split, BM=512):
    def body(e_ref, gs_ref, d0, d1, ap, b_ref, th_ref, x_ref, ss_ref):
        degt = d0[...] + d1[...] + ap[...]
        dis = jnp.where(degt > 0, lax.rsqrt(degt), 0.0)
        if feat_split:
            esum = jnp.concatenate([e_ref[0], e_ref[1]], axis=1)
            gs = jnp.concatenate([gs_ref[0], gs_ref[1]], axis=1)
        else:
            esum = (e_ref[0] + e_ref[1])[:, :D]
            gs = gs_ref[...][:, :D]
        xv = jnp.maximum(dis * (esum + ap[...] * gs) + b_ref[...], 0.0) * ap[...]
        x_ref[...] = xv
        ss_ref[...] = jnp.dot(xv, th_ref[...], preferred_element_type=F32) * dis

    gs_spec = (pl.BlockSpec((2, BM, DW), lambda i: (0, i, 0)) if feat_split
               else pl.BlockSpec((BM, DW), lambda i: (i, 0)))
    return pl.pallas_call(
        body,
        grid=(NP // BM,),
        in_specs=[
            pl.BlockSpec((2, BM, DW), lambda i: (0, i, 0)),
            gs_spec,
            pl.BlockSpec((BM, 1), lambda i: (i, 0)),
            pl.BlockSpec((BM, 1), lambda i: (i, 0)),
            pl.BlockSpec((BM, 1), lambda i: (i, 0)),
            pl.BlockSpec((1, D), lambda i: (0, 0)),
            pl.BlockSpec((D, 1), lambda i: (0, 0)),
        ],
        out_specs=[
            pl.BlockSpec((BM, D), lambda i: (i, 0)),
            pl.BlockSpec((BM, 1), lambda i: (i, 0)),
        ],
        out_shape=[
            jax.ShapeDtypeStruct((NP, D), F32),
            jax.ShapeDtypeStruct((NP, 1), F32),
        ],
    )


def _tc_select(kk, D, last):
    def body(*refs):
        if last:
            (zp_ref, d0, d1, ap, ss, x_ref, ro1, ro2,
             wr1, br1, wr2, br2, wr3, br3, wo, bo,
             alive_ref, xp_ref, ro_ref, out_ref) = refs
        else:
            (zp_ref, d0, d1, ap, ss, x_ref,
             alive_ref, xp_ref, ro_ref) = refs
        degt = d0[...] + d1[...] + ap[...]
        dis = jnp.where(degt > 0, lax.rsqrt(degt), 0.0)
        z = dis * (zp_ref[0] + zp_ref[1] + ap[...] * ss[...])
        zkey = jnp.where(ap[...] > 0, z, NEG)
        ki = lax.bitcast_convert_type(zkey, I32)
        ks = jnp.where(ki < 0, ki ^ jnp.int32(0x7FFFFFFF), ki)

        def bstep(t, pre):
            cand = pre + lax.shift_left(jnp.int32(1), jnp.int32(31) - t)
            cnt = jnp.sum((ks >= cand).astype(I32))
            return jnp.where(cnt >= kk, cand, pre)
        T = lax.fori_loop(0, 32, bstep, jnp.int32(-2147483648))

        gt = ks > T
        need = jnp.int32(kk) - jnp.sum(gt.astype(I32))
        tied = ks == T
        idx = (lax.broadcasted_iota(I32, (80, 128), 0) * 128
               + lax.broadcasted_iota(I32, (80, 128), 1))

        def tstep(t, pre):
            cand = pre + lax.shift_left(jnp.int32(1), jnp.int32(13) - t)
            cnt = jnp.sum((tied & (idx < cand)).astype(I32))
            return jnp.where(cnt < need, cand, pre)
        C0 = lax.fori_loop(0, 14, tstep, jnp.int32(0))

        alive = gt | (tied & (idx < (C0 + 1)) & (need > 0))
        a = alive.astype(F32)
        alive_ref[...] = a
        gate = a * jnp.tanh(z)
        xp = x_ref[...] * gate[..., None]
        xp_ref[...] = xp
        ro = jnp.sum(xp, axis=(0, 1)).reshape(1, D)
        ro_ref[...] = ro
        if last:
            r = (jnp.dot(ro1[...], wr1[...], preferred_element_type=F32) + br1[...]
                 + jnp.dot(ro2[...], wr2[...], preferred_element_type=F32) + br2[...]
                 + jnp.dot(ro, wr3[...], preferred_element_type=F32) + br3[...])
            out_ref[...] = (jnp.dot(jnp.maximum(r, 0.0), wo[...],
                                    preferred_element_type=F32) + bo[...])

    outs = [
        jax.ShapeDtypeStruct((80, 128), F32),
        jax.ShapeDtypeStruct((80, 128, D), F32),
        jax.ShapeDtypeStruct((1, D), F32),
    ]
    if last:
        outs.append(jax.ShapeDtypeStruct((1, 128), F32))
    return pl.pallas_call(body, out_shape=outs)


def kernel(x, edge_index, edge_attr, batch, W1, b1, theta1, W2, b2, theta2,
           W3, b3, theta3, Wr1, br1, Wr2, br2, Wr3, br3, Wout, bout):
    del batch
    row = edge_index[0]
    col = edge_index[1]
    pe = EP - E
    rowf = jnp.concatenate([row, jnp.full((pe,), NP - 1, I32)])
    colf = jnp.concatenate([col, jnp.full((pe,), NP - 1, I32)])
    wf = jnp.concatenate([edge_attr.astype(F32), jnp.zeros((pe,), F32)])
    rowp = rowf.reshape(EP // CH, CH)
    colp = colf.reshape(EP // CH, CH)
    wp = wf.reshape(EP // CH, CH)
    rowp64 = rowf.reshape(EP // 64, 64)
    colp64 = colf.reshape(EP // 64, 64)
    wp64 = wf.reshape(EP // 64, 64)
    xpad = jnp.pad(x, ((0, NP - N), (0, 0)))
    alive0 = jnp.concatenate([jnp.ones((N,), F32), jnp.zeros((NP - N,), F32)])

    deg_k = _sc_scalar_prop(True, False)
    zprop_w = _sc_scalar_prop(False, True)
    zprop = _sc_scalar_prop(False, False)

    def layer(inp, alive, K, D, W, b, theta, kk, first, last, head=None):
        feat_split = first
        acol = alive.reshape(NP, 1)
        a2d = alive.reshape(80, 128)
        degp = deg_k(rowp, colp, alive).reshape(2, NP)
        d0 = degp[0].reshape(NP, 1)
        d1 = degp[1].reshape(NP, 1)
        d02 = degp[0].reshape(80, 128)
        d12 = degp[1].reshape(80, 128)
        gs = _tc_mm(K, D, feat_split)(inp, W, d0, d1, acol)
        feat = _sc_feat_prop(feat_split, first)
        gs_sc = gs.reshape(2 * NP, DW) if feat_split else gs
        if first:
            es = feat(gs_sc, rowp64, colp64, wp64)
        else:
            es = feat(gs_sc, rowp64, colp64)
        xv, ss = _tc_epi(D, feat_split)(es.reshape(2, NP, DW), gs, d0, d1,
                                        acol, b.reshape(1, D), theta)
        if first:
            zp = zprop_w(rowp, colp, ss.reshape(NP), wp)
        else:
            zp = zprop(rowp, colp, ss.reshape(NP))
        sel_in = [zp.reshape(2, 80, 128), d02, d12, a2d,
                  ss.reshape(80, 128), xv.reshape(80, 128, D)]
        if last:
            sel_in += head
            return _tc_select(kk, D, True)(*sel_in)
        return _tc_select(kk, D, False)(*sel_in)

    a1, x1p, ro1 = layer(xpad, alive0, 256, 256, W1, b1, theta1,
                         K1, True, False)
    a2, x2p, ro2 = layer(x1p.reshape(NP, 256), a1.reshape(NP), 256, 128,
                         W2, b2, theta2, K2, False, False)
    head = [ro1, ro2, Wr1, br1.reshape(1, 64), Wr2, br2.reshape(1, 64),
            Wr3, br3.reshape(1, 64), Wout, bout.reshape(1, 128)]
    _, _, _, out = layer(x2p.reshape(NP, 128), a2.reshape(NP), 128, 64,
                         W3, b3, theta3, K3, False, True, head)
    return out
